# Initial kernel scaffold; baseline (speedup 1.0000x reference)
import jax
import jax.numpy as jnp
from jax.experimental import pallas as pl


def kernel(events):
    raise NotImplementedError("write your pallas kernel here")


# speedup vs baseline: 1.3337x; 1.3337x over previous
import functools

import jax
import jax.numpy as jnp
from jax import lax
from jax.experimental import pallas as pl
from jax.experimental.pallas import tpu as pltpu
from jax.experimental.pallas import tpu_sc as plsc

N_EV = 4194304
COLS = 1024
ROWS = (N_EV * 4) // COLS
GRID_STEPS = 16
RPB = ROWS // GRID_STEPS

NUM_SLICES = 10
DOWN_H, DOWN_W = 180, 320
GRID_N = NUM_SLICES * 2 * DOWN_H * DOWN_W

NW = 32
PER_TILE = N_EV // NW
CHUNK = 8192
NCHUNK = PER_TILE // CHUNK
ZSLICE = GRID_N // 16


def _minmax_body(ev_ref, tmin_ref, tmax_ref):
    i = pl.program_id(0)
    v = ev_ref[...]
    col = lax.broadcasted_iota(jnp.int32, v.shape, 1)
    is_t = (col & 3) == 2
    tm = jnp.min(jnp.where(is_t, v, jnp.inf))
    tM = jnp.max(jnp.where(is_t, v, -jnp.inf))

    @pl.when(i == 0)
    def _():
        tmin_ref[...] = jnp.full((8, 128), tm, jnp.float32)
        tmax_ref[...] = jnp.full((8, 128), tM, jnp.float32)

    @pl.when(i > 0)
    def _():
        tmin_ref[...] = jnp.minimum(tmin_ref[...], tm)
        tmax_ref[...] = jnp.maximum(tmax_ref[...], tM)


def _merge_body(a_ref, b_ref, o_ref):
    o_ref[...] = jnp.maximum(a_ref[...], b_ref[...])


def _scatter_body(ev_hbm, mn_hbm, mx_hbm, g0, g1, ev_v, idx_v, ones_v, mn_v, mx_v, sem):
    cid = lax.axis_index("c")
    sid = lax.axis_index("s")

    pltpu.sync_copy(mn_hbm.at[pl.ds(0, 16)], mn_v)
    pltpu.sync_copy(mx_hbm.at[pl.ds(0, 16)], mx_v)
    mn = mn_v[...]
    mx = mx_v[...]
    cond = mx > mn
    denom = jnp.where(cond, mx - mn, jnp.float32(1.0))

    zeros16 = jnp.zeros((16,), jnp.float32)
    ones16 = jnp.full((16,), 1.0, jnp.float32)
    iota16 = lax.iota(jnp.int32, 16)
    p0 = iota16 * 4
    p1 = p0 + 1
    p2 = p0 + 2
    p3 = p0 + 3

    def fill_zeros(i, c):
        ones_v[pl.ds(i * 16, 16)] = zeros16
        return c

    lax.fori_loop(0, CHUNK // 16, fill_zeros, 0)

    def run(g):
        zbase = sid * ZSLICE
        zsrc = ones_v
        for k in range(8):
            pltpu.sync_copy(zsrc, g.at[pl.ds(zbase + k * CHUNK, CHUNK)])
        pltpu.sync_copy(
            zsrc.at[pl.ds(0, ZSLICE - 8 * CHUNK)],
            g.at[pl.ds(zbase + 8 * CHUNK, ZSLICE - 8 * CHUNK)],
        )
        plsc.subcore_barrier()

        def fill_ones(i, c):
            ones_v[pl.ds(i * 16, 16)] = ones16
            return c

        lax.fori_loop(0, CHUNK // 16, fill_ones, 0)

        tbase4 = (cid * 16 + sid) * PER_TILE * 4

        def chunk(j, c):
            pltpu.sync_copy(ev_hbm.at[pl.ds(tbase4 + j * (CHUNK * 4), CHUNK * 4)], ev_v)

            def ev16(i, c2):
                base = i * 64
                xs = plsc.load_gather(ev_v, [p0 + base])
                ys = plsc.load_gather(ev_v, [p1 + base])
                ts = plsc.load_gather(ev_v, [p2 + base])
                ps = plsc.load_gather(ev_v, [p3 + base])
                tsn = jnp.where(cond, (ts - mn) / denom * 50.0, ts)
                si = jnp.clip((tsn / 5.0).astype(jnp.int32), 0, NUM_SLICES - 1)
                xc = (xs / 4.0).astype(jnp.int32)
                yc = (ys / 4.0).astype(jnp.int32)
                ch = si * 2 + (ps <= 0.0).astype(jnp.int32)
                idx_v[pl.ds(i * 16, 16)] = ch * (DOWN_H * DOWN_W) + yc * DOWN_W + xc
                return c2

            lax.fori_loop(0, CHUNK // 16, ev16, 0)
            pltpu.async_copy(ones_v, g.at[idx_v], sem).wait()
            return c

        lax.fori_loop(0, NCHUNK, chunk, 0)

    @pl.when(cid == 0)
    def _():
        run(g0)

    @pl.when(cid == 1)
    def _():
        run(g1)


@functools.cache
def _build_scatter_kernel():
    mesh = plsc.VectorSubcoreMesh(core_axis_name="c", subcore_axis_name="s")
    return pl.kernel(
        _scatter_body,
        out_type=[jax.ShapeDtypeStruct((GRID_N,), jnp.float32)] * 2,
        mesh=mesh,
        scratch_types=[
            pltpu.VMEM((CHUNK * 4,), jnp.float32),
            pltpu.VMEM((CHUNK,), jnp.int32),
            pltpu.VMEM((CHUNK,), jnp.float32),
            pltpu.VMEM((16,), jnp.float32),
            pltpu.VMEM((16,), jnp.float32),
            pltpu.SemaphoreType.DMA,
        ],
        compiler_params=pltpu.CompilerParams(needs_layout_passes=False),
    )


def kernel(events):
    ev_flat = events.reshape(ROWS, COLS)
    tmin_b, tmax_b = pl.pallas_call(
        _minmax_body,
        grid=(GRID_STEPS,),
        in_specs=[pl.BlockSpec((RPB, COLS), lambda i: (i, 0))],
        out_shape=[jax.ShapeDtypeStruct((8, 128), jnp.float32)] * 2,
        out_specs=[
            pl.BlockSpec((8, 128), lambda i: (0, 0)),
            pl.BlockSpec((8, 128), lambda i: (0, 0)),
        ],
    )(ev_flat)
    g0, g1 = _build_scatter_kernel()(
        events.reshape(N_EV * 4), tmin_b.reshape(1024), tmax_b.reshape(1024)
    )
    merged = pl.pallas_call(
        _merge_body,
        grid=(1,),
        in_specs=[
            pl.BlockSpec((GRID_N // 128, 128), lambda i: (0, 0)),
            pl.BlockSpec((GRID_N // 128, 128), lambda i: (0, 0)),
        ],
        out_shape=jax.ShapeDtypeStruct((GRID_N // 128, 128), jnp.float32),
        out_specs=pl.BlockSpec((GRID_N // 128, 128), lambda i: (0, 0)),
    )(g0.reshape(GRID_N // 128, 128), g1.reshape(GRID_N // 128, 128))
    return merged.reshape(NUM_SLICES * 2, DOWN_H, DOWN_W)


# speedup vs baseline: 1.5643x; 1.1729x over previous
import functools

import jax
import jax.numpy as jnp
from jax import lax
from jax.experimental import pallas as pl
from jax.experimental.pallas import tpu as pltpu
from jax.experimental.pallas import tpu_sc as plsc

N_EV = 4194304

NUM_SLICES = 10
DOWN_H, DOWN_W = 180, 320
GRID_N = NUM_SLICES * 2 * DOWN_H * DOWN_W

NW = 32
PER_TILE = N_EV // NW
CHUNK = 8192
NCHUNK = PER_TILE // CHUNK
ZSLICE = GRID_N // 16


def _mm_body(ev_hbm, mn_out, mx_out, ev_a, ev_b, t_v, sem_a, sem_b):
    cid = lax.axis_index("c")
    sid = lax.axis_index("s")
    wid = cid * 16 + sid
    ebase4 = wid * PER_TILE * 4
    iota16 = lax.iota(jnp.int32, 16)
    p2 = iota16 * 4 + 2

    bufs = (ev_a, ev_b)
    sems = (sem_a, sem_b)
    h = [None, None]
    h[0] = pltpu.async_copy(ev_hbm.at[pl.ds(ebase4, CHUNK * 4)], ev_a, sem_a)
    carry = (jnp.full((16,), jnp.inf, jnp.float32), jnp.full((16,), -jnp.inf, jnp.float32))
    for j in range(NCHUNK):
        b = j % 2
        if j + 1 < NCHUNK:
            h[1 - b] = pltpu.async_copy(
                ev_hbm.at[pl.ds(ebase4 + (j + 1) * CHUNK * 4, CHUNK * 4)], bufs[1 - b], sems[1 - b]
            )
        h[b].wait()
        ev = bufs[b]

        @plsc.parallel_loop(0, CHUNK // 16, unroll=8, carry=carry)
        def loop(i, c):
            ts = plsc.load_gather(ev, [p2 + i * 64])
            return (jnp.minimum(c[0], ts), jnp.maximum(c[1], ts))

        carry = loop

    t_v[...] = carry[0]
    pltpu.sync_copy(t_v, mn_out.at[wid])
    t_v[...] = carry[1]
    pltpu.sync_copy(t_v, mx_out.at[wid])


def _mm_reduce_body(mn_ref, mx_ref, mnb_ref, mxb_ref):
    mnb_ref[...] = jnp.full((8, 128), jnp.min(mn_ref[...]), jnp.float32)
    mxb_ref[...] = jnp.full((8, 128), jnp.max(mx_ref[...]), jnp.float32)


def _merge_body(a_ref, b_ref, o_ref):
    o_ref[...] = jnp.maximum(a_ref[...], b_ref[...])


def _scatter_body(ev_hbm, mn_hbm, mx_hbm, g0, g1, ev_a, ev_b, idx_a, ones_v,
                  mn_v, mx_v, sem_a, sem_b, ssem_a):
    cid = lax.axis_index("c")
    sid = lax.axis_index("s")
    wid = cid * 16 + sid

    pltpu.sync_copy(mn_hbm.at[pl.ds(0, 16)], mn_v)
    pltpu.sync_copy(mx_hbm.at[pl.ds(0, 16)], mx_v)
    mn = mn_v[...]
    mx = mx_v[...]
    cond = mx > mn
    denom = jnp.where(cond, mx - mn, jnp.float32(1.0))

    zeros16 = jnp.zeros((16,), jnp.float32)
    ones16 = jnp.full((16,), 1.0, jnp.float32)
    iota16 = lax.iota(jnp.int32, 16)
    p0 = iota16 * 4
    p1 = p0 + 1
    p2 = p0 + 2
    p3 = p0 + 3

    def fill_zeros(i, c):
        ones_v[pl.ds(i * 16, 16)] = zeros16
        return c

    lax.fori_loop(0, CHUNK // 16, fill_zeros, 0)

    def run(g):
        zbase = sid * ZSLICE
        for k in range(8):
            pltpu.sync_copy(ones_v, g.at[pl.ds(zbase + k * CHUNK, CHUNK)])
        pltpu.sync_copy(
            ones_v.at[pl.ds(0, ZSLICE - 8 * CHUNK)],
            g.at[pl.ds(zbase + 8 * CHUNK, ZSLICE - 8 * CHUNK)],
        )
        plsc.subcore_barrier()

        def fill_ones(i, c):
            ones_v[pl.ds(i * 16, 16)] = ones16
            return c

        lax.fori_loop(0, CHUNK // 16, fill_ones, 0)

        ebase4 = wid * PER_TILE * 4
        ev_bufs = (ev_a, ev_b)
        sems = (sem_a, sem_b)
        h = [None, None]
        hs = None
        h[0] = pltpu.async_copy(ev_hbm.at[pl.ds(ebase4, CHUNK * 4)], ev_a, sem_a)
        for j in range(NCHUNK):
            b = j % 2
            if j + 1 < NCHUNK:
                h[1 - b] = pltpu.async_copy(
                    ev_hbm.at[pl.ds(ebase4 + (j + 1) * CHUNK * 4, CHUNK * 4)],
                    ev_bufs[1 - b], sems[1 - b],
                )
            h[b].wait()
            if hs is not None:
                hs.wait()
            ev = ev_bufs[b]
            idx = idx_a

            @plsc.parallel_loop(0, CHUNK // 16, unroll=8)
            def loop(i):
                base = i * 64
                xs = plsc.load_gather(ev, [p0 + base])
                ys = plsc.load_gather(ev, [p1 + base])
                ts = plsc.load_gather(ev, [p2 + base])
                ps = plsc.load_gather(ev, [p3 + base])
                tsn = jnp.where(cond, (ts - mn) / denom * 50.0, ts)
                si = jnp.clip((tsn / 5.0).astype(jnp.int32), 0, NUM_SLICES - 1)
                xc = (xs / 4.0).astype(jnp.int32)
                yc = (ys / 4.0).astype(jnp.int32)
                ch = si * 2 + (ps <= 0.0).astype(jnp.int32)
                idx[pl.ds(i * 16, 16)] = ch * (DOWN_H * DOWN_W) + yc * DOWN_W + xc

            hs = pltpu.async_copy(ones_v, g.at[idx], ssem_a)
        hs.wait()

    @pl.when(cid == 0)
    def _():
        run(g0)

    @pl.when(cid == 1)
    def _():
        run(g1)


@functools.cache
def _build_sc_kernels():
    mesh = plsc.VectorSubcoreMesh(core_axis_name="c", subcore_axis_name="s")
    mm = pl.kernel(
        _mm_body,
        out_type=[jax.ShapeDtypeStruct((NW, 16), jnp.float32)] * 2,
        mesh=mesh,
        scratch_types=[
            pltpu.VMEM((CHUNK * 4,), jnp.float32),
            pltpu.VMEM((CHUNK * 4,), jnp.float32),
            pltpu.VMEM((16,), jnp.float32),
            pltpu.SemaphoreType.DMA,
            pltpu.SemaphoreType.DMA,
        ],
        compiler_params=pltpu.CompilerParams(needs_layout_passes=False, use_tc_tiling_on_sc=False),
    )
    scat = pl.kernel(
        _scatter_body,
        out_type=[jax.ShapeDtypeStruct((GRID_N,), jnp.float32)] * 2,
        mesh=mesh,
        scratch_types=[
            pltpu.VMEM((CHUNK * 4,), jnp.float32),
            pltpu.VMEM((CHUNK * 4,), jnp.float32),
            pltpu.VMEM((CHUNK,), jnp.int32),
            pltpu.VMEM((CHUNK,), jnp.float32),
            pltpu.VMEM((16,), jnp.float32),
            pltpu.VMEM((16,), jnp.float32),
            pltpu.SemaphoreType.DMA,
            pltpu.SemaphoreType.DMA,
            pltpu.SemaphoreType.DMA,
        ],
        compiler_params=pltpu.CompilerParams(needs_layout_passes=False, use_tc_tiling_on_sc=False),
    )
    return mm, scat


def kernel(events):
    mm, scat = _build_sc_kernels()
    ev_flat = events.reshape(N_EV * 4)
    mn_p, mx_p = mm(ev_flat)
    mn_b, mx_b = pl.pallas_call(
        _mm_reduce_body,
        in_specs=[
            pl.BlockSpec((NW, 16), lambda: (0, 0)),
            pl.BlockSpec((NW, 16), lambda: (0, 0)),
        ],
        out_shape=[jax.ShapeDtypeStruct((8, 128), jnp.float32)] * 2,
        out_specs=[
            pl.BlockSpec((8, 128), lambda: (0, 0)),
            pl.BlockSpec((8, 128), lambda: (0, 0)),
        ],
    )(mn_p, mx_p)
    g0, g1 = scat(ev_flat, mn_b.reshape(1024), mx_b.reshape(1024))
    merged = pl.pallas_call(
        _merge_body,
        in_specs=[
            pl.BlockSpec((GRID_N // 128, 128), lambda: (0, 0)),
            pl.BlockSpec((GRID_N // 128, 128), lambda: (0, 0)),
        ],
        out_shape=jax.ShapeDtypeStruct((GRID_N // 128, 128), jnp.float32),
        out_specs=pl.BlockSpec((GRID_N // 128, 128), lambda: (0, 0)),
    )(g0.reshape(GRID_N // 128, 128), g1.reshape(GRID_N // 128, 128))
    return merged.reshape(NUM_SLICES * 2, DOWN_H, DOWN_W)


# speedup vs baseline: 3.5814x; 2.2895x over previous
import functools

import jax
import jax.numpy as jnp
from jax import lax
from jax.experimental import pallas as pl
from jax.experimental.pallas import tpu as pltpu
from jax.experimental.pallas import tpu_sc as plsc

N_EV = 4194304

NUM_SLICES = 10
DOWN_H, DOWN_W = 180, 320
GRID_N = NUM_SLICES * 2 * DOWN_H * DOWN_W

NW = 32
PER_TILE = N_EV // NW
CHUNK = 8192
NCHUNK = PER_TILE // CHUNK
ZSLICE = GRID_N // 16


def _mm_body(ts_hbm, mn_out, mx_out, ts_a, ts_b, t_v, sem_a, sem_b):
    cid = lax.axis_index("c")
    sid = lax.axis_index("s")
    wid = cid * 16 + sid
    ebase = wid * PER_TILE

    bufs = (ts_a, ts_b)
    sems = (sem_a, sem_b)
    h = [None, None]
    h[0] = pltpu.async_copy(ts_hbm.at[pl.ds(ebase, CHUNK)], ts_a, sem_a)
    carry = (jnp.full((16,), jnp.inf, jnp.float32), jnp.full((16,), -jnp.inf, jnp.float32))
    for j in range(NCHUNK):
        b = j % 2
        if j + 1 < NCHUNK:
            h[1 - b] = pltpu.async_copy(
                ts_hbm.at[pl.ds(ebase + (j + 1) * CHUNK, CHUNK)], bufs[1 - b], sems[1 - b]
            )
        h[b].wait()
        buf = bufs[b]

        @plsc.parallel_loop(0, CHUNK // 16, unroll=8, carry=carry)
        def loop(i, c):
            ts = buf[pl.ds(i * 16, 16)]
            return (jnp.minimum(c[0], ts), jnp.maximum(c[1], ts))

        carry = loop

    t_v[...] = carry[0]
    pltpu.sync_copy(t_v, mn_out.at[wid])
    t_v[...] = carry[1]
    pltpu.sync_copy(t_v, mx_out.at[wid])


def _mm_reduce_body(mn_ref, mx_ref, mnb_ref, mxb_ref):
    mnb_ref[...] = jnp.full((8, 128), jnp.min(mn_ref[...]), jnp.float32)
    mxb_ref[...] = jnp.full((8, 128), jnp.max(mx_ref[...]), jnp.float32)


def _merge_body(a_ref, b_ref, o_ref):
    o_ref[...] = jnp.maximum(a_ref[...], b_ref[...])


def _scatter_body(xs_hbm, ys_hbm, ts_hbm, ps_hbm, mn_hbm, mx_hbm, g0, g1,
                  bx_a, bx_b, by_a, by_b, bt_a, bt_b, bp_a, bp_b,
                  idx_a, idx_b, ones_v, mn_v, mx_v,
                  sem_a, sem_b, ssem_a, ssem_b):
    cid = lax.axis_index("c")
    sid = lax.axis_index("s")
    wid = cid * 16 + sid

    pltpu.sync_copy(mn_hbm.at[pl.ds(0, 16)], mn_v)
    pltpu.sync_copy(mx_hbm.at[pl.ds(0, 16)], mx_v)
    mn = mn_v[...]
    mx = mx_v[...]
    cond = mx > mn
    denom = jnp.where(cond, mx - mn, jnp.float32(1.0))

    zeros16 = jnp.zeros((16,), jnp.float32)
    ones16 = jnp.full((16,), 1.0, jnp.float32)

    def fill_zeros(i, c):
        ones_v[pl.ds(i * 16, 16)] = zeros16
        return c

    lax.fori_loop(0, CHUNK // 16, fill_zeros, 0)

    def run(g):
        zbase = sid * ZSLICE
        for k in range(8):
            pltpu.sync_copy(ones_v, g.at[pl.ds(zbase + k * CHUNK, CHUNK)])
        pltpu.sync_copy(
            ones_v.at[pl.ds(0, ZSLICE - 8 * CHUNK)],
            g.at[pl.ds(zbase + 8 * CHUNK, ZSLICE - 8 * CHUNK)],
        )
        plsc.subcore_barrier()

        def fill_ones(i, c):
            ones_v[pl.ds(i * 16, 16)] = ones16
            return c

        lax.fori_loop(0, CHUNK // 16, fill_ones, 0)

        ebase = wid * PER_TILE
        xb = (bx_a, bx_b)
        yb = (by_a, by_b)
        tb = (bt_a, bt_b)
        pb = (bp_a, bp_b)
        ib = (idx_a, idx_b)
        sems = (sem_a, sem_b)
        ssems = (ssem_a, ssem_b)
        h = [None, None]
        hs = [None, None]

        def issue_reads(j, b):
            sl = pl.ds(ebase + j * CHUNK, CHUNK)
            return (
                pltpu.async_copy(xs_hbm.at[sl], xb[b], sems[b]),
                pltpu.async_copy(ys_hbm.at[sl], yb[b], sems[b]),
                pltpu.async_copy(ts_hbm.at[sl], tb[b], sems[b]),
                pltpu.async_copy(ps_hbm.at[sl], pb[b], sems[b]),
            )

        h[0] = issue_reads(0, 0)
        for j in range(NCHUNK):
            b = j % 2
            if j + 1 < NCHUNK:
                h[1 - b] = issue_reads(j + 1, 1 - b)
            for hh in h[b]:
                hh.wait()
            if hs[b] is not None:
                hs[b].wait()
            bx, by, bt, bp = xb[b], yb[b], tb[b], pb[b]
            idx = ib[b]

            @plsc.parallel_loop(0, CHUNK // 16, unroll=8)
            def loop(i):
                sl = pl.ds(i * 16, 16)
                xv = bx[sl]
                yv = by[sl]
                tv = bt[sl]
                pv = bp[sl]
                tsn = jnp.where(cond, (tv - mn) / denom * 50.0, tv)
                si = jnp.clip((tsn / 5.0).astype(jnp.int32), 0, NUM_SLICES - 1)
                xc = (xv / 4.0).astype(jnp.int32)
                yc = (yv / 4.0).astype(jnp.int32)
                ch = si * 2 + (pv <= 0.0).astype(jnp.int32)
                idx[sl] = ch * (DOWN_H * DOWN_W) + yc * DOWN_W + xc

            hs[b] = pltpu.async_copy(ones_v, g.at[idx], ssems[b])
        for hh in hs:
            if hh is not None:
                hh.wait()

    @pl.when(cid == 0)
    def _():
        run(g0)

    @pl.when(cid == 1)
    def _():
        run(g1)


@functools.cache
def _build_sc_kernels():
    mesh = plsc.VectorSubcoreMesh(core_axis_name="c", subcore_axis_name="s")
    mm = pl.kernel(
        _mm_body,
        out_type=[jax.ShapeDtypeStruct((NW, 16), jnp.float32)] * 2,
        mesh=mesh,
        scratch_types=[
            pltpu.VMEM((CHUNK,), jnp.float32),
            pltpu.VMEM((CHUNK,), jnp.float32),
            pltpu.VMEM((16,), jnp.float32),
            pltpu.SemaphoreType.DMA,
            pltpu.SemaphoreType.DMA,
        ],
        compiler_params=pltpu.CompilerParams(needs_layout_passes=False, use_tc_tiling_on_sc=False),
    )
    scat = pl.kernel(
        _scatter_body,
        out_type=[jax.ShapeDtypeStruct((GRID_N,), jnp.float32)] * 2,
        mesh=mesh,
        scratch_types=[
            pltpu.VMEM((CHUNK,), jnp.float32),
            pltpu.VMEM((CHUNK,), jnp.float32),
            pltpu.VMEM((CHUNK,), jnp.float32),
            pltpu.VMEM((CHUNK,), jnp.float32),
            pltpu.VMEM((CHUNK,), jnp.float32),
            pltpu.VMEM((CHUNK,), jnp.float32),
            pltpu.VMEM((CHUNK,), jnp.float32),
            pltpu.VMEM((CHUNK,), jnp.float32),
            pltpu.VMEM((CHUNK,), jnp.int32),
            pltpu.VMEM((CHUNK,), jnp.int32),
            pltpu.VMEM((CHUNK,), jnp.float32),
            pltpu.VMEM((16,), jnp.float32),
            pltpu.VMEM((16,), jnp.float32),
            pltpu.SemaphoreType.DMA,
            pltpu.SemaphoreType.DMA,
            pltpu.SemaphoreType.DMA,
            pltpu.SemaphoreType.DMA,
        ],
        compiler_params=pltpu.CompilerParams(needs_layout_passes=False, use_tc_tiling_on_sc=False),
    )
    return mm, scat


def kernel(events):
    mm, scat = _build_sc_kernels()
    xs = events[:, 0]
    ys = events[:, 1]
    ts = events[:, 2]
    ps = events[:, 3]
    mn_p, mx_p = mm(ts)
    mn_b, mx_b = pl.pallas_call(
        _mm_reduce_body,
        in_specs=[
            pl.BlockSpec((NW, 16), lambda: (0, 0)),
            pl.BlockSpec((NW, 16), lambda: (0, 0)),
        ],
        out_shape=[jax.ShapeDtypeStruct((8, 128), jnp.float32)] * 2,
        out_specs=[
            pl.BlockSpec((8, 128), lambda: (0, 0)),
            pl.BlockSpec((8, 128), lambda: (0, 0)),
        ],
    )(mn_p, mx_p)
    g0, g1 = scat(xs, ys, ts, ps, mn_b.reshape(1024), mx_b.reshape(1024))
    merged = pl.pallas_call(
        _merge_body,
        in_specs=[
            pl.BlockSpec((GRID_N // 128, 128), lambda: (0, 0)),
            pl.BlockSpec((GRID_N // 128, 128), lambda: (0, 0)),
        ],
        out_shape=jax.ShapeDtypeStruct((GRID_N // 128, 128), jnp.float32),
        out_specs=pl.BlockSpec((GRID_N // 128, 128), lambda: (0, 0)),
    )(g0.reshape(GRID_N // 128, 128), g1.reshape(GRID_N // 128, 128))
    return merged.reshape(NUM_SLICES * 2, DOWN_H, DOWN_W)


# speedup vs baseline: 29.5142x; 8.2410x over previous
import functools

import jax
import jax.numpy as jnp
from jax import lax
from jax.experimental import pallas as pl
from jax.experimental.pallas import tpu as pltpu
from jax.experimental.pallas import tpu_sc as plsc

N_EV = 4194304

NUM_SLICES = 10
DOWN_H, DOWN_W = 180, 320
GRID_N = NUM_SLICES * 2 * DOWN_H * DOWN_W

NW = 32
PER_TILE = N_EV // NW
CHUNK = 8192
NCHUNK = PER_TILE // CHUNK
ZB = GRID_N // NW
CHUNKB = 32768
NCHUNKB = N_EV // CHUNKB


def _mm_body(ts_hbm, mn_out, mx_out, ts_a, ts_b, t_v, sem_a, sem_b):
    cid = lax.axis_index("c")
    sid = lax.axis_index("s")
    wid = cid * 16 + sid
    ebase = wid * PER_TILE

    bufs = (ts_a, ts_b)
    sems = (sem_a, sem_b)
    h = [None, None]
    h[0] = pltpu.async_copy(ts_hbm.at[pl.ds(ebase, CHUNK)], ts_a, sem_a)
    carry = (jnp.full((16,), jnp.inf, jnp.float32), jnp.full((16,), -jnp.inf, jnp.float32))
    for j in range(NCHUNK):
        b = j % 2
        if j + 1 < NCHUNK:
            h[1 - b] = pltpu.async_copy(
                ts_hbm.at[pl.ds(ebase + (j + 1) * CHUNK, CHUNK)], bufs[1 - b], sems[1 - b]
            )
        h[b].wait()
        buf = bufs[b]

        @plsc.parallel_loop(0, CHUNK // 16, unroll=8, carry=carry)
        def loop(i, c):
            ts = buf[pl.ds(i * 16, 16)]
            return (jnp.minimum(c[0], ts), jnp.maximum(c[1], ts))

        carry = loop

    t_v[...] = carry[0]
    pltpu.sync_copy(t_v, mn_out.at[wid])
    t_v[...] = carry[1]
    pltpu.sync_copy(t_v, mx_out.at[wid])


def _mm_reduce_body(mn_ref, mx_ref, mnb_ref, mxb_ref):
    mnb_ref[...] = jnp.full((8, 128), jnp.min(mn_ref[...]), jnp.float32)
    mxb_ref[...] = jnp.full((8, 128), jnp.max(mx_ref[...]), jnp.float32)


def _idx_body(xs_hbm, ys_hbm, ts_hbm, ps_hbm, mn_hbm, mx_hbm, idx_out,
              bx_a, bx_b, by_a, by_b, bt_a, bt_b, bp_a, bp_b,
              io_a, io_b, mn_v, mx_v, sem_a, sem_b, osem_a, osem_b):
    cid = lax.axis_index("c")
    sid = lax.axis_index("s")
    wid = cid * 16 + sid

    pltpu.sync_copy(mn_hbm.at[pl.ds(0, 16)], mn_v)
    pltpu.sync_copy(mx_hbm.at[pl.ds(0, 16)], mx_v)
    mn = mn_v[...]
    mx = mx_v[...]
    cond = mx > mn
    denom = jnp.where(cond, mx - mn, jnp.float32(1.0))

    ebase = wid * PER_TILE
    xb = (bx_a, bx_b)
    yb = (by_a, by_b)
    tb = (bt_a, bt_b)
    pb = (bp_a, bp_b)
    ob = (io_a, io_b)
    sems = (sem_a, sem_b)
    osems = (osem_a, osem_b)
    h = [None, None]
    ho = [None, None]

    def issue_reads(j, b):
        sl = pl.ds(ebase + j * CHUNK, CHUNK)
        return (
            pltpu.async_copy(xs_hbm.at[sl], xb[b], sems[b]),
            pltpu.async_copy(ys_hbm.at[sl], yb[b], sems[b]),
            pltpu.async_copy(ts_hbm.at[sl], tb[b], sems[b]),
            pltpu.async_copy(ps_hbm.at[sl], pb[b], sems[b]),
        )

    h[0] = issue_reads(0, 0)
    for j in range(NCHUNK):
        b = j % 2
        if j + 1 < NCHUNK:
            h[1 - b] = issue_reads(j + 1, 1 - b)
        for hh in h[b]:
            hh.wait()
        if ho[b] is not None:
            ho[b].wait()
        bx, by, bt, bp = xb[b], yb[b], tb[b], pb[b]
        idx = ob[b]

        @plsc.parallel_loop(0, CHUNK // 16, unroll=8)
        def loop(i):
            sl = pl.ds(i * 16, 16)
            xv = bx[sl]
            yv = by[sl]
            tv = bt[sl]
            pv = bp[sl]
            tsn = jnp.where(cond, (tv - mn) / denom * 50.0, tv)
            si = jnp.clip((tsn / 5.0).astype(jnp.int32), 0, NUM_SLICES - 1)
            xc = (xv / 4.0).astype(jnp.int32)
            yc = (yv / 4.0).astype(jnp.int32)
            ch = si * 2 + (pv <= 0.0).astype(jnp.int32)
            idx[sl] = ch * (DOWN_H * DOWN_W) + yc * DOWN_W + xc

        ho[b] = pltpu.async_copy(idx, idx_out.at[pl.ds(ebase + j * CHUNK, CHUNK)], osems[b])
    for hh in ho:
        if hh is not None:
            hh.wait()


def _bin_body(idx_hbm, grid_out, ia, ib, gslice, sem_a, sem_b):
    cid = lax.axis_index("c")
    sid = lax.axis_index("s")
    wid = cid * 16 + sid
    lo = wid * ZB

    zeros16 = jnp.zeros((16,), jnp.float32)
    ones16 = jnp.full((16,), 1.0, jnp.float32)

    @plsc.parallel_loop(0, (ZB + 15) // 16, unroll=8)
    def zero(i):
        gslice[pl.ds(i * 16, 16)] = zeros16

    bufs = (ia, ib)
    sems = (sem_a, sem_b)
    h = [None, None]
    h[0] = pltpu.async_copy(idx_hbm.at[pl.ds(0, CHUNKB)], ia, sem_a)
    for j in range(NCHUNKB):
        b = j % 2
        if j + 1 < NCHUNKB:
            h[1 - b] = pltpu.async_copy(
                idx_hbm.at[pl.ds((j + 1) * CHUNKB, CHUNKB)], bufs[1 - b], sems[1 - b]
            )
        h[b].wait()
        buf = bufs[b]

        @plsc.parallel_loop(0, CHUNKB // 16, unroll=8)
        def loop(i):
            iv = buf[pl.ds(i * 16, 16)] - lo
            m = (iv >= 0) & (iv < ZB)
            plsc.store_scatter(gslice, [iv], ones16, mask=m)

    pltpu.sync_copy(gslice.at[pl.ds(0, ZB)], grid_out.at[pl.ds(lo, ZB)])


@functools.cache
def _build_sc_kernels():
    mesh = plsc.VectorSubcoreMesh(core_axis_name="c", subcore_axis_name="s")
    params = pltpu.CompilerParams(needs_layout_passes=False, use_tc_tiling_on_sc=False)
    mm = pl.kernel(
        _mm_body,
        out_type=[jax.ShapeDtypeStruct((NW, 16), jnp.float32)] * 2,
        mesh=mesh,
        scratch_types=[
            pltpu.VMEM((CHUNK,), jnp.float32),
            pltpu.VMEM((CHUNK,), jnp.float32),
            pltpu.VMEM((16,), jnp.float32),
            pltpu.SemaphoreType.DMA,
            pltpu.SemaphoreType.DMA,
        ],
        compiler_params=params,
    )
    idxk = pl.kernel(
        _idx_body,
        out_type=jax.ShapeDtypeStruct((N_EV,), jnp.int32),
        mesh=mesh,
        scratch_types=[
            pltpu.VMEM((CHUNK,), jnp.float32),
            pltpu.VMEM((CHUNK,), jnp.float32),
            pltpu.VMEM((CHUNK,), jnp.float32),
            pltpu.VMEM((CHUNK,), jnp.float32),
            pltpu.VMEM((CHUNK,), jnp.float32),
            pltpu.VMEM((CHUNK,), jnp.float32),
            pltpu.VMEM((CHUNK,), jnp.float32),
            pltpu.VMEM((CHUNK,), jnp.float32),
            pltpu.VMEM((CHUNK,), jnp.int32),
            pltpu.VMEM((CHUNK,), jnp.int32),
            pltpu.VMEM((16,), jnp.float32),
            pltpu.VMEM((16,), jnp.float32),
            pltpu.SemaphoreType.DMA,
            pltpu.SemaphoreType.DMA,
            pltpu.SemaphoreType.DMA,
            pltpu.SemaphoreType.DMA,
        ],
        compiler_params=params,
    )
    bink = pl.kernel(
        _bin_body,
        out_type=jax.ShapeDtypeStruct((GRID_N,), jnp.float32),
        mesh=mesh,
        scratch_types=[
            pltpu.VMEM((CHUNKB,), jnp.int32),
            pltpu.VMEM((CHUNKB,), jnp.int32),
            pltpu.VMEM((ZB,), jnp.float32),
            pltpu.SemaphoreType.DMA,
            pltpu.SemaphoreType.DMA,
        ],
        compiler_params=params,
    )
    return mm, idxk, bink


def kernel(events):
    mm, idxk, bink = _build_sc_kernels()
    xs = events[:, 0]
    ys = events[:, 1]
    ts = events[:, 2]
    ps = events[:, 3]
    mn_p, mx_p = mm(ts)
    mn_b, mx_b = pl.pallas_call(
        _mm_reduce_body,
        in_specs=[
            pl.BlockSpec((NW, 16), lambda: (0, 0)),
            pl.BlockSpec((NW, 16), lambda: (0, 0)),
        ],
        out_shape=[jax.ShapeDtypeStruct((8, 128), jnp.float32)] * 2,
        out_specs=[
            pl.BlockSpec((8, 128), lambda: (0, 0)),
            pl.BlockSpec((8, 128), lambda: (0, 0)),
        ],
    )(mn_p, mx_p)
    idx = idxk(xs, ys, ts, ps, mn_b.reshape(1024), mx_b.reshape(1024))
    grid = bink(idx)
    return grid.reshape(NUM_SLICES * 2, DOWN_H, DOWN_W)


# speedup vs baseline: 29.8756x; 1.0122x over previous
import functools

import jax
import jax.numpy as jnp
from jax import lax
from jax.experimental import pallas as pl
from jax.experimental.pallas import tpu as pltpu
from jax.experimental.pallas import tpu_sc as plsc

N_EV = 4194304

NUM_SLICES = 10
DOWN_H, DOWN_W = 180, 320
GRID_N = NUM_SLICES * 2 * DOWN_H * DOWN_W

NW = 32
PER_TILE = N_EV // NW
CHUNK = 8192
NCHUNK = PER_TILE // CHUNK
ZB = GRID_N // NW
CHUNKB = 32768
NCHUNKB = N_EV // CHUNKB


def _mm_body(ts_hbm, mn_out, mx_out, ts_a, ts_b, t_v, sem_a, sem_b):
    cid = lax.axis_index("c")
    sid = lax.axis_index("s")
    wid = cid * 16 + sid
    ebase = wid * PER_TILE

    bufs = (ts_a, ts_b)
    sems = (sem_a, sem_b)
    h = [None, None]
    h[0] = pltpu.async_copy(ts_hbm.at[pl.ds(ebase, CHUNK)], ts_a, sem_a)
    carry = (jnp.full((16,), jnp.inf, jnp.float32), jnp.full((16,), -jnp.inf, jnp.float32))
    for j in range(NCHUNK):
        b = j % 2
        if j + 1 < NCHUNK:
            h[1 - b] = pltpu.async_copy(
                ts_hbm.at[pl.ds(ebase + (j + 1) * CHUNK, CHUNK)], bufs[1 - b], sems[1 - b]
            )
        h[b].wait()
        buf = bufs[b]

        @plsc.parallel_loop(0, CHUNK // 16, unroll=8, carry=carry)
        def loop(i, c):
            ts = buf[pl.ds(i * 16, 16)]
            return (jnp.minimum(c[0], ts), jnp.maximum(c[1], ts))

        carry = loop

    t_v[...] = carry[0]
    pltpu.sync_copy(t_v, mn_out.at[pl.ds(wid * 16, 16)])
    t_v[...] = carry[1]
    pltpu.sync_copy(t_v, mx_out.at[pl.ds(wid * 16, 16)])


def _idx_body(xs_hbm, ys_hbm, ts_hbm, ps_hbm, mn_hbm, mx_hbm, idx_out,
              bx_a, bx_b, by_a, by_b, bt_a, bt_b, bp_a, bp_b,
              io_a, io_b, mn_v, mx_v, sem_a, sem_b, osem_a, osem_b):
    cid = lax.axis_index("c")
    sid = lax.axis_index("s")
    wid = cid * 16 + sid

    pltpu.sync_copy(mn_hbm, mn_v)
    pltpu.sync_copy(mx_hbm, mx_v)
    mn = mn_v[pl.ds(0, 16)]
    mx = mx_v[pl.ds(0, 16)]
    for k in range(1, NW):
        mn = jnp.minimum(mn, mn_v[pl.ds(k * 16, 16)])
        mx = jnp.maximum(mx, mx_v[pl.ds(k * 16, 16)])
    mn = jnp.zeros((16,), jnp.float32) + lax.reduce_min(mn, (0,))
    mx = jnp.zeros((16,), jnp.float32) + lax.reduce_max(mx, (0,))
    cond = mx > mn
    denom = jnp.where(cond, mx - mn, jnp.float32(1.0))

    ebase = wid * PER_TILE
    xb = (bx_a, bx_b)
    yb = (by_a, by_b)
    tb = (bt_a, bt_b)
    pb = (bp_a, bp_b)
    ob = (io_a, io_b)
    sems = (sem_a, sem_b)
    osems = (osem_a, osem_b)
    h = [None, None]
    ho = [None, None]

    def issue_reads(j, b):
        sl = pl.ds(ebase + j * CHUNK, CHUNK)
        return (
            pltpu.async_copy(xs_hbm.at[sl], xb[b], sems[b]),
            pltpu.async_copy(ys_hbm.at[sl], yb[b], sems[b]),
            pltpu.async_copy(ts_hbm.at[sl], tb[b], sems[b]),
            pltpu.async_copy(ps_hbm.at[sl], pb[b], sems[b]),
        )

    h[0] = issue_reads(0, 0)
    for j in range(NCHUNK):
        b = j % 2
        if j + 1 < NCHUNK:
            h[1 - b] = issue_reads(j + 1, 1 - b)
        for hh in h[b]:
            hh.wait()
        if ho[b] is not None:
            ho[b].wait()
        bx, by, bt, bp = xb[b], yb[b], tb[b], pb[b]
        idx = ob[b]

        @plsc.parallel_loop(0, CHUNK // 16, unroll=8)
        def loop(i):
            sl = pl.ds(i * 16, 16)
            xv = bx[sl]
            yv = by[sl]
            tv = bt[sl]
            pv = bp[sl]
            tsn = jnp.where(cond, (tv - mn) / denom * 50.0, tv)
            si = jnp.clip((tsn / 5.0).astype(jnp.int32), 0, NUM_SLICES - 1)
            xc = (xv / 4.0).astype(jnp.int32)
            yc = (yv / 4.0).astype(jnp.int32)
            ch = si * 2 + (pv <= 0.0).astype(jnp.int32)
            idx[sl] = ch * (DOWN_H * DOWN_W) + yc * DOWN_W + xc

        ho[b] = pltpu.async_copy(idx, idx_out.at[pl.ds(ebase + j * CHUNK, CHUNK)], osems[b])
    for hh in ho:
        if hh is not None:
            hh.wait()


def _bin_body(idx_hbm, grid_out, ia, ib, gslice, sem_a, sem_b):
    cid = lax.axis_index("c")
    sid = lax.axis_index("s")
    wid = cid * 16 + sid
    lo = wid * ZB

    zeros16 = jnp.zeros((16,), jnp.float32)
    ones16 = jnp.full((16,), 1.0, jnp.float32)

    @plsc.parallel_loop(0, (ZB + 15) // 16, unroll=8)
    def zero(i):
        gslice[pl.ds(i * 16, 16)] = zeros16

    bufs = (ia, ib)
    sems = (sem_a, sem_b)
    h = [None, None]
    h[0] = pltpu.async_copy(idx_hbm.at[pl.ds(0, CHUNKB)], ia, sem_a)
    for j in range(NCHUNKB):
        b = j % 2
        if j + 1 < NCHUNKB:
            h[1 - b] = pltpu.async_copy(
                idx_hbm.at[pl.ds((j + 1) * CHUNKB, CHUNKB)], bufs[1 - b], sems[1 - b]
            )
        h[b].wait()
        buf = bufs[b]

        zb_u = jnp.full((16,), ZB, jnp.uint32)

        @plsc.parallel_loop(0, CHUNKB // 16, unroll=8)
        def loop(i):
            iv = buf[pl.ds(i * 16, 16)] - lo
            m = plsc.bitcast(iv, jnp.uint32) < zb_u
            plsc.store_scatter(gslice, [iv], ones16, mask=m)

    pltpu.sync_copy(gslice.at[pl.ds(0, ZB)], grid_out.at[pl.ds(lo, ZB)])


@functools.cache
def _build_sc_kernels():
    mesh = plsc.VectorSubcoreMesh(core_axis_name="c", subcore_axis_name="s")
    params = pltpu.CompilerParams(needs_layout_passes=False, use_tc_tiling_on_sc=False)
    mm = pl.kernel(
        _mm_body,
        out_type=[jax.ShapeDtypeStruct((NW * 16,), jnp.float32)] * 2,
        mesh=mesh,
        scratch_types=[
            pltpu.VMEM((CHUNK,), jnp.float32),
            pltpu.VMEM((CHUNK,), jnp.float32),
            pltpu.VMEM((16,), jnp.float32),
            pltpu.SemaphoreType.DMA,
            pltpu.SemaphoreType.DMA,
        ],
        compiler_params=params,
    )
    idxk = pl.kernel(
        _idx_body,
        out_type=jax.ShapeDtypeStruct((N_EV,), jnp.int32),
        mesh=mesh,
        scratch_types=[
            pltpu.VMEM((CHUNK,), jnp.float32),
            pltpu.VMEM((CHUNK,), jnp.float32),
            pltpu.VMEM((CHUNK,), jnp.float32),
            pltpu.VMEM((CHUNK,), jnp.float32),
            pltpu.VMEM((CHUNK,), jnp.float32),
            pltpu.VMEM((CHUNK,), jnp.float32),
            pltpu.VMEM((CHUNK,), jnp.float32),
            pltpu.VMEM((CHUNK,), jnp.float32),
            pltpu.VMEM((CHUNK,), jnp.int32),
            pltpu.VMEM((CHUNK,), jnp.int32),
            pltpu.VMEM((NW * 16,), jnp.float32),
            pltpu.VMEM((NW * 16,), jnp.float32),
            pltpu.SemaphoreType.DMA,
            pltpu.SemaphoreType.DMA,
            pltpu.SemaphoreType.DMA,
            pltpu.SemaphoreType.DMA,
        ],
        compiler_params=params,
    )
    bink = pl.kernel(
        _bin_body,
        out_type=jax.ShapeDtypeStruct((GRID_N,), jnp.float32),
        mesh=mesh,
        scratch_types=[
            pltpu.VMEM((CHUNKB,), jnp.int32),
            pltpu.VMEM((CHUNKB,), jnp.int32),
            pltpu.VMEM((ZB,), jnp.float32),
            pltpu.SemaphoreType.DMA,
            pltpu.SemaphoreType.DMA,
        ],
        compiler_params=params,
    )
    return mm, idxk, bink


def kernel(events):
    mm, idxk, bink = _build_sc_kernels()
    xs = events[:, 0]
    ys = events[:, 1]
    ts = events[:, 2]
    ps = events[:, 3]
    mn_p, mx_p = mm(ts)
    idx = idxk(xs, ys, ts, ps, mn_p, mx_p)
    grid = bink(idx)
    return grid.reshape(NUM_SLICES * 2, DOWN_H, DOWN_W)


# speedup vs baseline: 39.5898x; 1.3252x over previous
import functools

import jax
import jax.numpy as jnp
from jax import lax
from jax.experimental import pallas as pl
from jax.experimental.pallas import tpu as pltpu
from jax.experimental.pallas import tpu_sc as plsc

N_EV = 4194304

NUM_SLICES = 10
DOWN_H, DOWN_W = 180, 320
GRID_N = NUM_SLICES * 2 * DOWN_H * DOWN_W

NW = 32
PER_TILE = N_EV // NW
CHUNK = 8192
NCHUNK = PER_TILE // CHUNK
ZB = GRID_N // 16
CHUNKB = 16384
NCHUNKB = (N_EV // 2) // CHUNKB


def _mm_body(ts_hbm, mn_out, mx_out, ts_a, ts_b, t_v, sem_a, sem_b):
    cid = lax.axis_index("c")
    sid = lax.axis_index("s")
    wid = cid * 16 + sid
    ebase = wid * PER_TILE

    bufs = (ts_a, ts_b)
    sems = (sem_a, sem_b)
    h = [None, None]
    h[0] = pltpu.async_copy(ts_hbm.at[pl.ds(ebase, CHUNK)], ts_a, sem_a)
    carry = (jnp.full((16,), jnp.inf, jnp.float32), jnp.full((16,), -jnp.inf, jnp.float32))
    for j in range(NCHUNK):
        b = j % 2
        if j + 1 < NCHUNK:
            h[1 - b] = pltpu.async_copy(
                ts_hbm.at[pl.ds(ebase + (j + 1) * CHUNK, CHUNK)], bufs[1 - b], sems[1 - b]
            )
        h[b].wait()
        buf = bufs[b]

        @plsc.parallel_loop(0, CHUNK // 16, unroll=8, carry=carry)
        def loop(i, c):
            ts = buf[pl.ds(i * 16, 16)]
            return (jnp.minimum(c[0], ts), jnp.maximum(c[1], ts))

        carry = loop

    t_v[...] = carry[0]
    pltpu.sync_copy(t_v, mn_out.at[pl.ds(wid * 16, 16)])
    t_v[...] = carry[1]
    pltpu.sync_copy(t_v, mx_out.at[pl.ds(wid * 16, 16)])


def _idx_body(xs_hbm, ys_hbm, ts_hbm, ps_hbm, mn_hbm, mx_hbm, idx_out,
              bx_a, bx_b, by_a, by_b, bt_a, bt_b, bp_a, bp_b,
              io_a, io_b, mn_v, mx_v, sem_a, sem_b, osem_a, osem_b):
    cid = lax.axis_index("c")
    sid = lax.axis_index("s")
    wid = cid * 16 + sid

    pltpu.sync_copy(mn_hbm, mn_v)
    pltpu.sync_copy(mx_hbm, mx_v)
    mn = mn_v[pl.ds(0, 16)]
    mx = mx_v[pl.ds(0, 16)]
    for k in range(1, NW):
        mn = jnp.minimum(mn, mn_v[pl.ds(k * 16, 16)])
        mx = jnp.maximum(mx, mx_v[pl.ds(k * 16, 16)])
    mn = jnp.zeros((16,), jnp.float32) + lax.reduce_min(mn, (0,))
    mx = jnp.zeros((16,), jnp.float32) + lax.reduce_max(mx, (0,))
    cond = mx > mn
    denom = jnp.where(cond, mx - mn, jnp.float32(1.0))

    ebase = wid * PER_TILE
    xb = (bx_a, bx_b)
    yb = (by_a, by_b)
    tb = (bt_a, bt_b)
    pb = (bp_a, bp_b)
    ob = (io_a, io_b)
    sems = (sem_a, sem_b)
    osems = (osem_a, osem_b)
    h = [None, None]
    ho = [None, None]

    def issue_reads(j, b):
        sl = pl.ds(ebase + j * CHUNK, CHUNK)
        return (
            pltpu.async_copy(xs_hbm.at[sl], xb[b], sems[b]),
            pltpu.async_copy(ys_hbm.at[sl], yb[b], sems[b]),
            pltpu.async_copy(ts_hbm.at[sl], tb[b], sems[b]),
            pltpu.async_copy(ps_hbm.at[sl], pb[b], sems[b]),
        )

    h[0] = issue_reads(0, 0)
    for j in range(NCHUNK):
        b = j % 2
        if j + 1 < NCHUNK:
            h[1 - b] = issue_reads(j + 1, 1 - b)
        for hh in h[b]:
            hh.wait()
        if ho[b] is not None:
            ho[b].wait()
        bx, by, bt, bp = xb[b], yb[b], tb[b], pb[b]
        idx = ob[b]

        @plsc.parallel_loop(0, CHUNK // 16, unroll=8)
        def loop(i):
            sl = pl.ds(i * 16, 16)
            xv = bx[sl]
            yv = by[sl]
            tv = bt[sl]
            pv = bp[sl]
            tsn = jnp.where(cond, (tv - mn) / denom * 50.0, tv)
            si = jnp.clip((tsn / 5.0).astype(jnp.int32), 0, NUM_SLICES - 1)
            xc = (xv / 4.0).astype(jnp.int32)
            yc = (yv / 4.0).astype(jnp.int32)
            ch = si * 2 + (pv <= 0.0).astype(jnp.int32)
            idx[sl] = ch * (DOWN_H * DOWN_W) + yc * DOWN_W + xc

        ho[b] = pltpu.async_copy(idx, idx_out.at[pl.ds(ebase + j * CHUNK, CHUNK)], osems[b])
    for hh in ho:
        if hh is not None:
            hh.wait()


def _bin_body(idx_hbm, g0_out, g1_out, ia, ib, gslice, sem_a, sem_b):
    cid = lax.axis_index("c")
    sid = lax.axis_index("s")
    lo = sid * ZB
    ebase = cid * (N_EV // 2)

    zeros16 = jnp.zeros((16,), jnp.float32)
    ones16 = jnp.full((16,), 1.0, jnp.float32)
    zb_u = jnp.full((16,), ZB, jnp.uint32)

    @plsc.parallel_loop(0, ZB // 16, unroll=8)
    def zero(i):
        gslice[pl.ds(i * 16, 16)] = zeros16

    bufs = (ia, ib)
    sems = (sem_a, sem_b)
    h = [None, None]
    h[0] = pltpu.async_copy(idx_hbm.at[pl.ds(ebase, CHUNKB)], ia, sem_a)
    for j in range(NCHUNKB):
        b = j % 2
        if j + 1 < NCHUNKB:
            h[1 - b] = pltpu.async_copy(
                idx_hbm.at[pl.ds(ebase + (j + 1) * CHUNKB, CHUNKB)], bufs[1 - b], sems[1 - b]
            )
        h[b].wait()
        buf = bufs[b]

        @plsc.parallel_loop(0, CHUNKB // 16, unroll=8)
        def loop(i):
            iv = buf[pl.ds(i * 16, 16)] - lo
            m = plsc.bitcast(iv, jnp.uint32) < zb_u
            plsc.store_scatter(gslice, [iv], ones16, mask=m)

    @pl.when(cid == 0)
    def _():
        pltpu.sync_copy(gslice.at[pl.ds(0, ZB)], g0_out.at[pl.ds(lo, ZB)])

    @pl.when(cid == 1)
    def _():
        pltpu.sync_copy(gslice.at[pl.ds(0, ZB)], g1_out.at[pl.ds(lo, ZB)])


def _merge_body(a_ref, b_ref, o_ref):
    o_ref[...] = jnp.maximum(a_ref[...], b_ref[...])


@functools.cache
def _build_sc_kernels():
    mesh = plsc.VectorSubcoreMesh(core_axis_name="c", subcore_axis_name="s")
    params = pltpu.CompilerParams(needs_layout_passes=False, use_tc_tiling_on_sc=False)
    mm = pl.kernel(
        _mm_body,
        out_type=[jax.ShapeDtypeStruct((NW * 16,), jnp.float32)] * 2,
        mesh=mesh,
        scratch_types=[
            pltpu.VMEM((CHUNK,), jnp.float32),
            pltpu.VMEM((CHUNK,), jnp.float32),
            pltpu.VMEM((16,), jnp.float32),
            pltpu.SemaphoreType.DMA,
            pltpu.SemaphoreType.DMA,
        ],
        compiler_params=params,
    )
    idxk = pl.kernel(
        _idx_body,
        out_type=jax.ShapeDtypeStruct((N_EV,), jnp.int32),
        mesh=mesh,
        scratch_types=[
            pltpu.VMEM((CHUNK,), jnp.float32),
            pltpu.VMEM((CHUNK,), jnp.float32),
            pltpu.VMEM((CHUNK,), jnp.float32),
            pltpu.VMEM((CHUNK,), jnp.float32),
            pltpu.VMEM((CHUNK,), jnp.float32),
            pltpu.VMEM((CHUNK,), jnp.float32),
            pltpu.VMEM((CHUNK,), jnp.float32),
            pltpu.VMEM((CHUNK,), jnp.float32),
            pltpu.VMEM((CHUNK,), jnp.int32),
            pltpu.VMEM((CHUNK,), jnp.int32),
            pltpu.VMEM((NW * 16,), jnp.float32),
            pltpu.VMEM((NW * 16,), jnp.float32),
            pltpu.SemaphoreType.DMA,
            pltpu.SemaphoreType.DMA,
            pltpu.SemaphoreType.DMA,
            pltpu.SemaphoreType.DMA,
        ],
        compiler_params=params,
    )
    bink = pl.kernel(
        _bin_body,
        out_type=[jax.ShapeDtypeStruct((GRID_N,), jnp.float32)] * 2,
        mesh=mesh,
        scratch_types=[
            pltpu.VMEM((CHUNKB,), jnp.int32),
            pltpu.VMEM((CHUNKB,), jnp.int32),
            pltpu.VMEM((ZB,), jnp.float32),
            pltpu.SemaphoreType.DMA,
            pltpu.SemaphoreType.DMA,
        ],
        compiler_params=params,
    )
    return mm, idxk, bink


def kernel(events):
    mm, idxk, bink = _build_sc_kernels()
    xs = events[:, 0]
    ys = events[:, 1]
    ts = events[:, 2]
    ps = events[:, 3]
    mn_p, mx_p = mm(ts)
    idx = idxk(xs, ys, ts, ps, mn_p, mx_p)
    g0, g1 = bink(idx)
    merged = pl.pallas_call(
        _merge_body,
        in_specs=[
            pl.BlockSpec((GRID_N // 128, 128), lambda: (0, 0)),
            pl.BlockSpec((GRID_N // 128, 128), lambda: (0, 0)),
        ],
        out_shape=jax.ShapeDtypeStruct((GRID_N // 128, 128), jnp.float32),
        out_specs=pl.BlockSpec((GRID_N // 128, 128), lambda: (0, 0)),
    )(g0.reshape(GRID_N // 128, 128), g1.reshape(GRID_N // 128, 128))
    return merged.reshape(NUM_SLICES * 2, DOWN_H, DOWN_W)


# speedup vs baseline: 45.5196x; 1.1498x over previous
import functools

import jax
import jax.numpy as jnp
from jax import lax
from jax.experimental import pallas as pl
from jax.experimental.pallas import tpu as pltpu
from jax.experimental.pallas import tpu_sc as plsc

N_EV = 4194304

NUM_SLICES = 10
DOWN_H, DOWN_W = 180, 320
GRID_N = NUM_SLICES * 2 * DOWN_H * DOWN_W

NW = 32
PER_TILE = N_EV // NW
CHUNK = 8192
NCHUNK = PER_TILE // CHUNK
ZB = GRID_N // 16
CHUNKB = 16384
NCHUNKB = (N_EV // 2) // CHUNKB


def _mm_body(ts_hbm, mn_out, mx_out, ts_a, ts_b, t_v, sem_a, sem_b):
    cid = lax.axis_index("c")
    sid = lax.axis_index("s")
    wid = cid * 16 + sid
    ebase = 2 * N_EV + wid * PER_TILE

    bufs = (ts_a, ts_b)
    sems = (sem_a, sem_b)
    h = [None, None]
    h[0] = pltpu.async_copy(ts_hbm.at[pl.ds(ebase, CHUNK)], ts_a, sem_a)
    carry = (jnp.full((16,), jnp.inf, jnp.float32), jnp.full((16,), -jnp.inf, jnp.float32))
    for j in range(NCHUNK):
        b = j % 2
        if j + 1 < NCHUNK:
            h[1 - b] = pltpu.async_copy(
                ts_hbm.at[pl.ds(ebase + (j + 1) * CHUNK, CHUNK)], bufs[1 - b], sems[1 - b]
            )
        h[b].wait()
        buf = bufs[b]

        @plsc.parallel_loop(0, CHUNK // 16, unroll=8, carry=carry)
        def loop(i, c):
            ts = buf[pl.ds(i * 16, 16)]
            return (jnp.minimum(c[0], ts), jnp.maximum(c[1], ts))

        carry = loop

    t_v[...] = carry[0]
    pltpu.sync_copy(t_v, mn_out.at[pl.ds(wid * 16, 16)])
    t_v[...] = carry[1]
    pltpu.sync_copy(t_v, mx_out.at[pl.ds(wid * 16, 16)])


def _idx_body(ev_hbm, mn_hbm, mx_hbm, idx_out,
              bx_a, bx_b, by_a, by_b, bt_a, bt_b, bp_a, bp_b,
              io_a, io_b, mn_v, mx_v, sem_a, sem_b, osem_a, osem_b):
    cid = lax.axis_index("c")
    sid = lax.axis_index("s")
    wid = cid * 16 + sid

    pltpu.sync_copy(mn_hbm, mn_v)
    pltpu.sync_copy(mx_hbm, mx_v)
    mn = mn_v[pl.ds(0, 16)]
    mx = mx_v[pl.ds(0, 16)]
    for k in range(1, NW):
        mn = jnp.minimum(mn, mn_v[pl.ds(k * 16, 16)])
        mx = jnp.maximum(mx, mx_v[pl.ds(k * 16, 16)])
    mn = jnp.zeros((16,), jnp.float32) + lax.reduce_min(mn, (0,))
    mx = jnp.zeros((16,), jnp.float32) + lax.reduce_max(mx, (0,))
    cond = mx > mn
    denom = jnp.where(cond, mx - mn, jnp.float32(1.0))

    ebase = wid * PER_TILE
    xb = (bx_a, bx_b)
    yb = (by_a, by_b)
    tb = (bt_a, bt_b)
    pb = (bp_a, bp_b)
    ob = (io_a, io_b)
    sems = (sem_a, sem_b)
    osems = (osem_a, osem_b)
    h = [None, None]
    ho = [None, None]

    def issue_reads(j, b):
        base = ebase + j * CHUNK
        return (
            pltpu.async_copy(ev_hbm.at[pl.ds(base, CHUNK)], xb[b], sems[b]),
            pltpu.async_copy(ev_hbm.at[pl.ds(N_EV + base, CHUNK)], yb[b], sems[b]),
            pltpu.async_copy(ev_hbm.at[pl.ds(2 * N_EV + base, CHUNK)], tb[b], sems[b]),
            pltpu.async_copy(ev_hbm.at[pl.ds(3 * N_EV + base, CHUNK)], pb[b], sems[b]),
        )

    h[0] = issue_reads(0, 0)
    for j in range(NCHUNK):
        b = j % 2
        if j + 1 < NCHUNK:
            h[1 - b] = issue_reads(j + 1, 1 - b)
        for hh in h[b]:
            hh.wait()
        if ho[b] is not None:
            ho[b].wait()
        bx, by, bt, bp = xb[b], yb[b], tb[b], pb[b]
        idx = ob[b]

        @plsc.parallel_loop(0, CHUNK // 16, unroll=8)
        def loop(i):
            sl = pl.ds(i * 16, 16)
            xv = bx[sl]
            yv = by[sl]
            tv = bt[sl]
            pv = bp[sl]
            tsn = jnp.where(cond, (tv - mn) / denom * 50.0, tv)
            si = jnp.clip((tsn / 5.0).astype(jnp.int32), 0, NUM_SLICES - 1)
            xc = (xv / 4.0).astype(jnp.int32)
            yc = (yv / 4.0).astype(jnp.int32)
            ch = si * 2 + (pv <= 0.0).astype(jnp.int32)
            idx[sl] = ch * (DOWN_H * DOWN_W) + yc * DOWN_W + xc

        ho[b] = pltpu.async_copy(idx, idx_out.at[pl.ds(ebase + j * CHUNK, CHUNK)], osems[b])
    for hh in ho:
        if hh is not None:
            hh.wait()


def _bin_body(idx_hbm, g0_out, g1_out, ia, ib, gslice, sem_a, sem_b):
    cid = lax.axis_index("c")
    sid = lax.axis_index("s")
    lo = sid * ZB
    ebase = cid * (N_EV // 2)

    zeros16 = jnp.zeros((16,), jnp.float32)
    ones16 = jnp.full((16,), 1.0, jnp.float32)
    zb_u = jnp.full((16,), ZB, jnp.uint32)

    @plsc.parallel_loop(0, ZB // 16, unroll=8)
    def zero(i):
        gslice[pl.ds(i * 16, 16)] = zeros16

    bufs = (ia, ib)
    sems = (sem_a, sem_b)
    h = [None, None]
    h[0] = pltpu.async_copy(idx_hbm.at[pl.ds(ebase, CHUNKB)], ia, sem_a)
    for j in range(NCHUNKB):
        b = j % 2
        if j + 1 < NCHUNKB:
            h[1 - b] = pltpu.async_copy(
                idx_hbm.at[pl.ds(ebase + (j + 1) * CHUNKB, CHUNKB)], bufs[1 - b], sems[1 - b]
            )
        h[b].wait()
        buf = bufs[b]

        @plsc.parallel_loop(0, CHUNKB // 16, unroll=8)
        def loop(i):
            iv = buf[pl.ds(i * 16, 16)] - lo
            m = plsc.bitcast(iv, jnp.uint32) < zb_u
            plsc.store_scatter(gslice, [iv], ones16, mask=m)

    @pl.when(cid == 0)
    def _():
        pltpu.sync_copy(gslice.at[pl.ds(0, ZB)], g0_out.at[pl.ds(lo, ZB)])

    @pl.when(cid == 1)
    def _():
        pltpu.sync_copy(gslice.at[pl.ds(0, ZB)], g1_out.at[pl.ds(lo, ZB)])


def _merge_body(a_ref, b_ref, o_ref):
    o_ref[...] = jnp.maximum(a_ref[...], b_ref[...])


@functools.cache
def _build_sc_kernels():
    mesh = plsc.VectorSubcoreMesh(core_axis_name="c", subcore_axis_name="s")
    params = pltpu.CompilerParams(needs_layout_passes=False, use_tc_tiling_on_sc=False)
    mm = pl.kernel(
        _mm_body,
        out_type=[jax.ShapeDtypeStruct((NW * 16,), jnp.float32)] * 2,
        mesh=mesh,
        scratch_types=[
            pltpu.VMEM((CHUNK,), jnp.float32),
            pltpu.VMEM((CHUNK,), jnp.float32),
            pltpu.VMEM((16,), jnp.float32),
            pltpu.SemaphoreType.DMA,
            pltpu.SemaphoreType.DMA,
        ],
        compiler_params=params,
    )
    idxk = pl.kernel(
        _idx_body,
        out_type=jax.ShapeDtypeStruct((N_EV,), jnp.int32),
        mesh=mesh,
        scratch_types=[
            pltpu.VMEM((CHUNK,), jnp.float32),
            pltpu.VMEM((CHUNK,), jnp.float32),
            pltpu.VMEM((CHUNK,), jnp.float32),
            pltpu.VMEM((CHUNK,), jnp.float32),
            pltpu.VMEM((CHUNK,), jnp.float32),
            pltpu.VMEM((CHUNK,), jnp.float32),
            pltpu.VMEM((CHUNK,), jnp.float32),
            pltpu.VMEM((CHUNK,), jnp.float32),
            pltpu.VMEM((CHUNK,), jnp.int32),
            pltpu.VMEM((CHUNK,), jnp.int32),
            pltpu.VMEM((NW * 16,), jnp.float32),
            pltpu.VMEM((NW * 16,), jnp.float32),
            pltpu.SemaphoreType.DMA,
            pltpu.SemaphoreType.DMA,
            pltpu.SemaphoreType.DMA,
            pltpu.SemaphoreType.DMA,
        ],
        compiler_params=params,
    )
    bink = pl.kernel(
        _bin_body,
        out_type=[jax.ShapeDtypeStruct((GRID_N,), jnp.float32)] * 2,
        mesh=mesh,
        scratch_types=[
            pltpu.VMEM((CHUNKB,), jnp.int32),
            pltpu.VMEM((CHUNKB,), jnp.int32),
            pltpu.VMEM((ZB,), jnp.float32),
            pltpu.SemaphoreType.DMA,
            pltpu.SemaphoreType.DMA,
        ],
        compiler_params=params,
    )
    return mm, idxk, bink


def kernel(events):
    mm, idxk, bink = _build_sc_kernels()
    ev4 = jnp.transpose(events).reshape(4 * N_EV)
    mn_p, mx_p = mm(ev4)
    idx = idxk(ev4, mn_p, mx_p)
    g0, g1 = bink(idx)
    merged = pl.pallas_call(
        _merge_body,
        in_specs=[
            pl.BlockSpec((GRID_N // 128, 128), lambda: (0, 0)),
            pl.BlockSpec((GRID_N // 128, 128), lambda: (0, 0)),
        ],
        out_shape=jax.ShapeDtypeStruct((GRID_N // 128, 128), jnp.float32),
        out_specs=pl.BlockSpec((GRID_N // 128, 128), lambda: (0, 0)),
    )(g0.reshape(GRID_N // 128, 128), g1.reshape(GRID_N // 128, 128))
    return merged.reshape(NUM_SLICES * 2, DOWN_H, DOWN_W)
